# baseline (speedup 1.0000x reference)
import functools

import jax
import jax.numpy as jnp
from jax import lax
from jax.experimental import pallas as pl
from jax.experimental.pallas import tpu as pltpu
from jax.experimental.pallas import tpu_sc as plsc

NC = 2
NS = 16
NW = NC * NS
CHUNK = 128
N_NODES = 10000
NPAD = 10016
RPS = NPAD // NS
SPMEM_WORDS = 2_000_000


@functools.cache
def _agg_kernel(C: int, nchunks: int, dtype=jnp.float32):
    elem_words = 1 if dtype == jnp.float32 else 0.5
    per_tile = (SPMEM_WORDS - int(NPAD * C * elem_words)) // NS \
        - 2 * nchunks * CHUNK
    NBUF = max(2, min(6, int(per_tile // (CHUNK * C * elem_words))))
    mesh = plsc.VectorSubcoreMesh(
        core_axis_name="c", subcore_axis_name="s", num_cores=NC, num_subcores=NS
    )

    @functools.partial(
        pl.kernel,
        out_type=jax.ShapeDtypeStruct((NC, NPAD, C), dtype),
        mesh=mesh,
        scratch_types=[
            pltpu.VMEM((nchunks, CHUNK), jnp.int32),
            pltpu.VMEM((nchunks, CHUNK), jnp.int32),
            pltpu.VMEM((NBUF, CHUNK, C), dtype),
            pltpu.VMEM_SHARED((NPAD, C), dtype),
        ] + [pltpu.SemaphoreType.DMA] * (2 * NBUF),
        compiler_params=pltpu.CompilerParams(use_tc_tiling_on_sc=False),
    )
    def agg(table, srci, dsti, zrows, out, idx_s, idx_d, rows_v, acc, *sems):
        gsems = sems[:NBUF]
        ssems = sems[NBUF:]
        c = lax.axis_index("c")
        s = lax.axis_index("s")
        w = c * NS + s

        pltpu.sync_copy(srci.at[w], idx_s)
        gd = {}
        sd = {}
        for k in range(min(NBUF, nchunks)):
            gd[k] = pltpu.async_copy(
                table.at[idx_s.at[k]], rows_v.at[k % NBUF], gsems[k % NBUF])
        pltpu.sync_copy(dsti.at[w], idx_d)
        pltpu.sync_copy(zrows, acc.at[pl.ds(s * RPS, RPS)])
        plsc.subcore_barrier()

        for k in range(nchunks):
            b = k % NBUF
            gd[k].wait()
            sd[k] = pltpu.async_copy(
                rows_v.at[b], acc.at[idx_d.at[k]], ssems[b], add=True)
            j = k - 1 + NBUF
            if k >= 1 and j < nchunks:
                sd[k - 1].wait()
                gd[j] = pltpu.async_copy(
                    table.at[idx_s.at[j]], rows_v.at[(k - 1) % NBUF],
                    gsems[(k - 1) % NBUF])
        for k in range(max(0, nchunks - NBUF), nchunks):
            sd[k].wait()

        plsc.subcore_barrier()
        pltpu.sync_copy(
            acc.at[pl.ds(s * RPS, RPS)], out.at[c, pl.ds(s * RPS, RPS)]
        )

    return agg


def _aggregate(table, srci, dsti, nchunks):
    C = table.shape[1]
    zrows = jnp.zeros((RPS, C), table.dtype)
    return _agg_kernel(C, nchunks, table.dtype)(table, srci, dsti, zrows)


def _dotT(a, w):
    return lax.dot_general(
        a, w, (((1,), (1,)), ((), ())),
        precision=lax.Precision.HIGHEST, preferred_element_type=jnp.float32,
    )


def _bn(h, g, b):
    m = jnp.mean(h, axis=0, keepdims=True)
    v = jnp.mean((h - m) ** 2, axis=0, keepdims=True)
    return (h - m) / jnp.sqrt(v + 1e-5) * g + b


def _tc(body, out_shape, *args):
    return pl.pallas_call(body, out_shape=out_shape)(*args)


def _shape(*s):
    return jax.ShapeDtypeStruct(s, jnp.float32)


_FBLK = 1000


def _finalize1(p160, in_c):
    def body(p_ref, a_ref, rdeg_ref):
        pa = p_ref[0].astype(jnp.float32)
        pb = p_ref[1].astype(jnp.float32)
        deg = pa[:, in_c:in_c + 1] + pb[:, in_c:in_c + 1]
        rdeg = 1.0 / jnp.maximum(deg, 1.0)
        rdeg_ref[...] = rdeg
        a_ref[...] = (pa[:, :in_c] + pb[:, :in_c]) * rdeg

    return pl.pallas_call(
        body,
        grid=(N_NODES // _FBLK,),
        in_specs=[pl.BlockSpec((2, _FBLK, in_c + 32), lambda i: (0, i, 0))],
        out_specs=[pl.BlockSpec((_FBLK, in_c), lambda i: (i, 0)),
                   pl.BlockSpec((_FBLK, 1), lambda i: (i, 0))],
        out_shape=[_shape(N_NODES, in_c), _shape(N_NODES, 1)],
    )(p160)


def _finalize(pAs, rdeg):
    widths = [p.shape[2] for p in pAs]
    C = sum(widths)

    def body(*refs):
        p_refs = refs[:len(pAs)]
        rdeg_ref = refs[len(pAs)]
        a_ref = refs[len(pAs) + 1]
        rd = rdeg_ref[...]
        off = 0
        for p_ref, w in zip(p_refs, widths):
            a_ref[:, off:off + w] = (p_ref[0].astype(jnp.float32)
                                     + p_ref[1].astype(jnp.float32)) * rd
            off += w

    return pl.pallas_call(
        body,
        grid=(N_NODES // _FBLK,),
        in_specs=[pl.BlockSpec((2, _FBLK, w), lambda i: (0, i, 0))
                  for w in widths]
        + [pl.BlockSpec((_FBLK, 1), lambda i: (i, 0))],
        out_specs=pl.BlockSpec((_FBLK, C), lambda i: (i, 0)),
        out_shape=_shape(N_NODES, C),
    )(*pAs, rdeg)


def kernel(x, edge_index, params):
    n = x.shape[0]
    e = edge_index.shape[1]
    in_c = x.shape[1]
    assert n == N_NODES

    ei = edge_index.astype(jnp.int32)
    nchunks = -(-e // (NW * CHUNK))
    epad = NW * nchunks * CHUNK
    srci = jnp.concatenate(
        [ei[0], jnp.zeros((epad - e,), jnp.int32)]).reshape(NW, nchunks, CHUNK)
    dsti = jnp.concatenate(
        [ei[1], jnp.full((epad - e,), n, jnp.int32)]).reshape(NW, nchunks, CHUNK)

    def agg(table):
        p = _aggregate(table.astype(jnp.bfloat16), srci, dsti, nchunks)
        return _finalize([p], rdeg)

    p = params
    eps = jax.random.normal(jax.random.key(42), (n, p["gcn_mean"][0].shape[0]),
                            dtype=jnp.float32)

    def r2(v):
        return v.reshape(1, -1)

    t160 = jnp.concatenate(
        [x.astype(jnp.bfloat16), jnp.ones((n, 32), jnp.bfloat16)], axis=1)
    p160 = _aggregate(t160, srci, dsti, nchunks)
    A1, rdeg = _finalize1(p160, in_c)

    def tc_conv_bn_relu(a_ref, xin_ref, Wl_ref, bl_ref, Wr_ref,
                        g_ref, b_ref, out_ref):
        h = (_dotT(a_ref[...], Wl_ref[...]) + bl_ref[...]
             + _dotT(xin_ref[...], Wr_ref[...]))
        out_ref[...] = jax.nn.relu(_bn(h, g_ref[...], b_ref[...]))

    def conv_bn_relu(a, xin, sage_p, bn_p):
        Wl_, bl_, Wr_ = sage_p
        g_, b_ = bn_p
        return _tc(tc_conv_bn_relu, _shape(n, Wl_.shape[0]),
                   a, xin, Wl_, r2(bl_), Wr_, r2(g_), r2(b_))

    def tc_conv_bn_add_relu(a_ref, xin_ref, skip_ref, Wl_ref,
                            bl_ref, Wr_ref, g_ref, b_ref, out_ref):
        h = (_dotT(a_ref[...], Wl_ref[...]) + bl_ref[...]
             + _dotT(xin_ref[...], Wr_ref[...]))
        out_ref[...] = jax.nn.relu(_bn(h, g_ref[...], b_ref[...]) + skip_ref[...])

    def conv_bn_add_relu(a, xin, skip, sage_p, bn_p):
        Wl_, bl_, Wr_ = sage_p
        g_, b_ = bn_p
        return _tc(tc_conv_bn_add_relu, _shape(n, Wl_.shape[0]),
                   a, xin, skip, Wl_, r2(bl_), Wr_, r2(g_), r2(b_))

    Wl, bl, Wr = p["enc_l1"]["conv1"]
    hid = Wl.shape[0]
    h1 = conv_bn_relu(A1, x, p["enc_l1"]["conv1"], p["enc_l1"]["norm1"])

    def tc2(a2_ref, a1_ref, h1_ref, x_ref,
            W2l_ref, b2_ref, W2r_ref, g2_ref, be2_ref,
            W3l_ref, b3_ref, W3r_ref, g3_ref, be3_ref, out_ref):
        h2 = (_dotT(a2_ref[...], W2l_ref[...]) + b2_ref[...]
              + _dotT(h1_ref[...], W2r_ref[...]))
        h2 = _bn(h2, g2_ref[...], be2_ref[...])
        x3 = (_dotT(a1_ref[...], W3l_ref[...]) + b3_ref[...]
              + _dotT(x_ref[...], W3r_ref[...]))
        x3 = _bn(x3, g3_ref[...], be3_ref[...])
        out_ref[...] = jax.nn.relu(h2 + x3)

    A2 = agg(h1)
    W2l, b2, W2r = p["enc_l1"]["conv2"]
    g2, be2 = p["enc_l1"]["norm2"]
    W3l, b3, W3r = p["enc_l1"]["conv3"]
    g3, be3 = p["enc_l1"]["norm3"]
    b1 = _tc(tc2, _shape(n, hid), A2, A1, h1, x,
             W2l, r2(b2), W2r, r2(g2), r2(be2),
             W3l, r2(b3), W3r, r2(g3), r2(be3))

    A3 = agg(b1)
    h3 = conv_bn_relu(A3, b1, p["enc_l2"]["conv1"], p["enc_l2"]["norm1"])
    A4 = agg(h3)
    b2n = conv_bn_add_relu(A4, h3, b1, p["enc_l2"]["conv2"], p["enc_l2"]["norm2"])

    A5 = agg(b2n)
    Wm, bm, Wrm = p["gcn_mean"]
    Ws, bs, Wrs = p["gcn_logstd"]

    def tc5(a_ref, b2_ref, Wm_ref, bm_ref, Wrm_ref,
            Ws_ref, bs_ref, Wrs_ref, eps_ref, z_ref, kl_ref):
        a = a_ref[...]
        b2v = b2_ref[...]
        mean = _dotT(a, Wm_ref[...]) + bm_ref[...] + _dotT(b2v, Wrm_ref[...])
        ls_raw = _dotT(a, Ws_ref[...]) + bs_ref[...] + _dotT(b2v, Wrs_ref[...])
        ls = jnp.clip(ls_raw, -10.0, 10.0)
        z_ref[...] = mean + eps_ref[...] * jnp.exp(ls)
        kl = -0.5 * jnp.mean(1.0 + ls_raw - mean ** 2 - jnp.exp(ls_raw))
        kl_ref[...] = kl.reshape(1, 1)

    out_c = Wm.shape[0]
    z, kl = _tc(tc5, [_shape(n, out_c), _shape(1, 1)],
                A5, b2n, Wm, r2(bm), Wrm, Ws, r2(bs), Wrs, eps)

    A6 = agg(z)
    Wd, bd, Wrd = p["dec_conv"]

    def tc6(a_ref, z_ref, Wd_ref, bd_ref, Wrd_ref, d_ref):
        d_ref[...] = (_dotT(a_ref[...], Wd_ref[...]) + bd_ref[...]
                      + _dotT(z_ref[...], Wrd_ref[...]))

    d = _tc(tc6, _shape(n, Wd.shape[0]), A6, z, Wd, r2(bd), Wrd)

    A7 = agg(d)
    h7 = conv_bn_relu(A7, d, p["dec_l1"]["conv1"], p["dec_l1"]["norm1"])
    A8 = agg(h7)
    b3n = conv_bn_add_relu(A8, h7, d, p["dec_l1"]["conv2"], p["dec_l1"]["norm2"])

    A9 = agg(b3n)
    h9 = conv_bn_relu(A9, b3n, p["dec_l2"]["conv1"], p["dec_l2"]["norm1"])
    A10 = agg(h9)

    Wf2l, bf2, Wf2r = p["dec_l2"]["conv2"]
    gf2, bef2 = p["dec_l2"]["norm2"]
    Wf3l, bf3, Wf3r = p["dec_l2"]["conv3"]
    gf3, bef3 = p["dec_l2"]["norm3"]

    def tc10(a10_ref, a9_ref, h9_ref, b3_ref, x_ref, kl_ref,
             W2l_ref, b2_ref, W2r_ref, g2_ref, be2_ref,
             W3l_ref, b3w_ref, W3r_ref, g3_ref, be3_ref,
             loss_ref, rl_ref):
        h2 = (_dotT(a10_ref[...], W2l_ref[...]) + b2_ref[...]
              + _dotT(h9_ref[...], W2r_ref[...]))
        h2 = _bn(h2, g2_ref[...], be2_ref[...])
        x3 = (_dotT(a9_ref[...], W3l_ref[...]) + b3w_ref[...]
              + _dotT(b3_ref[...], W3r_ref[...]))
        x3 = _bn(x3, g3_ref[...], be3_ref[...])
        dout = jax.nn.relu(h2 + x3)
        recon = jax.nn.sigmoid(dout)
        rc = jnp.clip(recon, 1e-7, 1.0 - 1e-7)
        xv = x_ref[...]
        rl = -jnp.mean(xv * jnp.log(rc) + (1.0 - xv) * jnp.log(1.0 - rc))
        rl_ref[...] = rl.reshape(1, 1)
        loss_ref[...] = rl.reshape(1, 1) + 0.2 * kl_ref[...]

    loss, recon_loss = _tc(
        tc10, [_shape(1, 1), _shape(1, 1)],
        A10, A9, h9, b3n, x, kl,
        Wf2l, r2(bf2), Wf2r, r2(gf2), r2(bef2),
        Wf3l, r2(bf3), Wf3r, r2(gf3), r2(bef3))

    return (loss.reshape(()), recon_loss.reshape(()), kl.reshape(()))


# speedup vs baseline: 1.0307x; 1.0307x over previous
import functools

import jax
import jax.numpy as jnp
from jax import lax
from jax.experimental import pallas as pl
from jax.experimental.pallas import tpu as pltpu
from jax.experimental.pallas import tpu_sc as plsc

NC = 2
NS = 16
NW = NC * NS
CHUNK = 128
N_NODES = 10000
NPAD = 10016
RPS = NPAD // NS
SPMEM_WORDS = 2_000_000


@functools.cache
def _agg_kernel(C: int, nchunks: int, dtype=jnp.float32):
    elem_words = 1 if dtype == jnp.float32 else 0.5
    per_tile = (SPMEM_WORDS - int(NPAD * C * elem_words)) // NS \
        - 2 * nchunks * CHUNK
    NBUF = max(2, min(6, int(per_tile // (CHUNK * C * elem_words))))
    mesh = plsc.VectorSubcoreMesh(
        core_axis_name="c", subcore_axis_name="s", num_cores=NC, num_subcores=NS
    )

    @functools.partial(
        pl.kernel,
        out_type=jax.ShapeDtypeStruct((NC, NPAD, C), dtype),
        mesh=mesh,
        scratch_types=[
            pltpu.VMEM((nchunks, CHUNK), jnp.int32),
            pltpu.VMEM((nchunks, CHUNK), jnp.int32),
            pltpu.VMEM((NBUF, CHUNK, C), dtype),
            pltpu.VMEM_SHARED((NPAD, C), dtype),
        ] + [pltpu.SemaphoreType.DMA] * (2 * NBUF),
        compiler_params=pltpu.CompilerParams(use_tc_tiling_on_sc=False),
    )
    def agg(table, srci, dsti, zrows, out, idx_s, idx_d, rows_v, acc, *sems):
        gsems = sems[:NBUF]
        ssems = sems[NBUF:]
        c = lax.axis_index("c")
        s = lax.axis_index("s")
        w = c * NS + s

        pltpu.sync_copy(srci.at[w], idx_s)
        gd = {}
        sd = {}
        for k in range(min(NBUF, nchunks)):
            gd[k] = pltpu.async_copy(
                table.at[idx_s.at[k]], rows_v.at[k % NBUF], gsems[k % NBUF])
        pltpu.sync_copy(dsti.at[w], idx_d)
        pltpu.sync_copy(zrows, acc.at[pl.ds(s * RPS, RPS)])
        plsc.subcore_barrier()

        for k in range(nchunks):
            b = k % NBUF
            gd[k].wait()
            sd[k] = pltpu.async_copy(
                rows_v.at[b], acc.at[idx_d.at[k]], ssems[b], add=True)
            j = k - 1 + NBUF
            if k >= 1 and j < nchunks:
                sd[k - 1].wait()
                gd[j] = pltpu.async_copy(
                    table.at[idx_s.at[j]], rows_v.at[(k - 1) % NBUF],
                    gsems[(k - 1) % NBUF])
        for k in range(max(0, nchunks - NBUF), nchunks):
            sd[k].wait()

        plsc.subcore_barrier()
        pltpu.sync_copy(
            acc.at[pl.ds(s * RPS, RPS)], out.at[c, pl.ds(s * RPS, RPS)]
        )

    return agg


def _aggregate(table, srci, dsti, nchunks):
    C = table.shape[1]
    zrows = jnp.zeros((RPS, C), table.dtype)
    return _agg_kernel(C, nchunks, table.dtype)(table, srci, dsti, zrows)


def _dotT(a, w):
    return lax.dot_general(
        a, w, (((1,), (1,)), ((), ())),
        precision=lax.Precision.HIGHEST, preferred_element_type=jnp.float32,
    )


def _bn(h, g, b):
    m = jnp.mean(h, axis=0, keepdims=True)
    v = jnp.mean((h - m) ** 2, axis=0, keepdims=True)
    return (h - m) / jnp.sqrt(v + 1e-5) * g + b


def _tc(body, out_shape, *args):
    return pl.pallas_call(body, out_shape=out_shape)(*args)


def _shape(*s):
    return jax.ShapeDtypeStruct(s, jnp.float32)


_FBLK = 1000


def _finalize1(p96, p64, in_c):
    def body(p96_ref, p64_ref, a_ref, rdeg_ref):
        pa = p96_ref[0].astype(jnp.float32)
        pb = p96_ref[1].astype(jnp.float32)
        deg = pa[:, 64:65] + pb[:, 64:65]
        rdeg = 1.0 / jnp.maximum(deg, 1.0)
        rdeg_ref[...] = rdeg
        a_ref[:, :64] = (pa[:, :64] + pb[:, :64]) * rdeg
        a_ref[:, 64:] = (p64_ref[0].astype(jnp.float32)
                         + p64_ref[1].astype(jnp.float32)) * rdeg

    return pl.pallas_call(
        body,
        grid=(N_NODES // _FBLK,),
        in_specs=[pl.BlockSpec((2, _FBLK, 96), lambda i: (0, i, 0)),
                  pl.BlockSpec((2, _FBLK, 64), lambda i: (0, i, 0))],
        out_specs=[pl.BlockSpec((_FBLK, in_c), lambda i: (i, 0)),
                   pl.BlockSpec((_FBLK, 1), lambda i: (i, 0))],
        out_shape=[_shape(N_NODES, in_c), _shape(N_NODES, 1)],
    )(p96, p64)


def _finalize(pAs, rdeg):
    widths = [p.shape[2] for p in pAs]
    C = sum(widths)

    def body(*refs):
        p_refs = refs[:len(pAs)]
        rdeg_ref = refs[len(pAs)]
        a_ref = refs[len(pAs) + 1]
        rd = rdeg_ref[...]
        off = 0
        for p_ref, w in zip(p_refs, widths):
            a_ref[:, off:off + w] = (p_ref[0].astype(jnp.float32)
                                     + p_ref[1].astype(jnp.float32)) * rd
            off += w

    return pl.pallas_call(
        body,
        grid=(N_NODES // _FBLK,),
        in_specs=[pl.BlockSpec((2, _FBLK, w), lambda i: (0, i, 0))
                  for w in widths]
        + [pl.BlockSpec((_FBLK, 1), lambda i: (i, 0))],
        out_specs=pl.BlockSpec((_FBLK, C), lambda i: (i, 0)),
        out_shape=_shape(N_NODES, C),
    )(*pAs, rdeg)


def kernel(x, edge_index, params):
    n = x.shape[0]
    e = edge_index.shape[1]
    in_c = x.shape[1]
    assert n == N_NODES

    ei = edge_index.astype(jnp.int32)
    nchunks = -(-e // (NW * CHUNK))
    epad = NW * nchunks * CHUNK
    srci = jnp.concatenate(
        [ei[0], jnp.zeros((epad - e,), jnp.int32)]).reshape(NW, nchunks, CHUNK)
    dsti = jnp.concatenate(
        [ei[1], jnp.full((epad - e,), n, jnp.int32)]).reshape(NW, nchunks, CHUNK)

    def agg(table):
        C = table.shape[1]
        t16 = table.astype(jnp.bfloat16)
        ps = [_aggregate(t16[:, i:min(i + 64, C)], srci, dsti, nchunks)
              for i in range(0, C, 64)]
        return _finalize(ps, rdeg)

    p = params
    eps = jax.random.normal(jax.random.key(42), (n, p["gcn_mean"][0].shape[0]),
                            dtype=jnp.float32)

    def r2(v):
        return v.reshape(1, -1)

    xb = x.astype(jnp.bfloat16)
    t96 = jnp.concatenate(
        [xb[:, :64], jnp.ones((n, 32), jnp.bfloat16)], axis=1)
    p96 = _aggregate(t96, srci, dsti, nchunks)
    p64 = _aggregate(xb[:, 64:in_c], srci, dsti, nchunks)
    A1, rdeg = _finalize1(p96, p64, in_c)

    def tc_conv_bn_relu(a_ref, xin_ref, Wl_ref, bl_ref, Wr_ref,
                        g_ref, b_ref, out_ref):
        h = (_dotT(a_ref[...], Wl_ref[...]) + bl_ref[...]
             + _dotT(xin_ref[...], Wr_ref[...]))
        out_ref[...] = jax.nn.relu(_bn(h, g_ref[...], b_ref[...]))

    def conv_bn_relu(a, xin, sage_p, bn_p):
        Wl_, bl_, Wr_ = sage_p
        g_, b_ = bn_p
        return _tc(tc_conv_bn_relu, _shape(n, Wl_.shape[0]),
                   a, xin, Wl_, r2(bl_), Wr_, r2(g_), r2(b_))

    def tc_conv_bn_add_relu(a_ref, xin_ref, skip_ref, Wl_ref,
                            bl_ref, Wr_ref, g_ref, b_ref, out_ref):
        h = (_dotT(a_ref[...], Wl_ref[...]) + bl_ref[...]
             + _dotT(xin_ref[...], Wr_ref[...]))
        out_ref[...] = jax.nn.relu(_bn(h, g_ref[...], b_ref[...]) + skip_ref[...])

    def conv_bn_add_relu(a, xin, skip, sage_p, bn_p):
        Wl_, bl_, Wr_ = sage_p
        g_, b_ = bn_p
        return _tc(tc_conv_bn_add_relu, _shape(n, Wl_.shape[0]),
                   a, xin, skip, Wl_, r2(bl_), Wr_, r2(g_), r2(b_))

    Wl, bl, Wr = p["enc_l1"]["conv1"]
    hid = Wl.shape[0]
    h1 = conv_bn_relu(A1, x, p["enc_l1"]["conv1"], p["enc_l1"]["norm1"])

    def tc2(a2_ref, a1_ref, h1_ref, x_ref,
            W2l_ref, b2_ref, W2r_ref, g2_ref, be2_ref,
            W3l_ref, b3_ref, W3r_ref, g3_ref, be3_ref, out_ref):
        h2 = (_dotT(a2_ref[...], W2l_ref[...]) + b2_ref[...]
              + _dotT(h1_ref[...], W2r_ref[...]))
        h2 = _bn(h2, g2_ref[...], be2_ref[...])
        x3 = (_dotT(a1_ref[...], W3l_ref[...]) + b3_ref[...]
              + _dotT(x_ref[...], W3r_ref[...]))
        x3 = _bn(x3, g3_ref[...], be3_ref[...])
        out_ref[...] = jax.nn.relu(h2 + x3)

    A2 = agg(h1)
    W2l, b2, W2r = p["enc_l1"]["conv2"]
    g2, be2 = p["enc_l1"]["norm2"]
    W3l, b3, W3r = p["enc_l1"]["conv3"]
    g3, be3 = p["enc_l1"]["norm3"]
    b1 = _tc(tc2, _shape(n, hid), A2, A1, h1, x,
             W2l, r2(b2), W2r, r2(g2), r2(be2),
             W3l, r2(b3), W3r, r2(g3), r2(be3))

    A3 = agg(b1)
    h3 = conv_bn_relu(A3, b1, p["enc_l2"]["conv1"], p["enc_l2"]["norm1"])
    A4 = agg(h3)
    b2n = conv_bn_add_relu(A4, h3, b1, p["enc_l2"]["conv2"], p["enc_l2"]["norm2"])

    A5 = agg(b2n)
    Wm, bm, Wrm = p["gcn_mean"]
    Ws, bs, Wrs = p["gcn_logstd"]

    def tc5(a_ref, b2_ref, Wm_ref, bm_ref, Wrm_ref,
            Ws_ref, bs_ref, Wrs_ref, eps_ref, z_ref, kl_ref):
        a = a_ref[...]
        b2v = b2_ref[...]
        mean = _dotT(a, Wm_ref[...]) + bm_ref[...] + _dotT(b2v, Wrm_ref[...])
        ls_raw = _dotT(a, Ws_ref[...]) + bs_ref[...] + _dotT(b2v, Wrs_ref[...])
        ls = jnp.clip(ls_raw, -10.0, 10.0)
        z_ref[...] = mean + eps_ref[...] * jnp.exp(ls)
        kl = -0.5 * jnp.mean(1.0 + ls_raw - mean ** 2 - jnp.exp(ls_raw))
        kl_ref[...] = kl.reshape(1, 1)

    out_c = Wm.shape[0]
    z, kl = _tc(tc5, [_shape(n, out_c), _shape(1, 1)],
                A5, b2n, Wm, r2(bm), Wrm, Ws, r2(bs), Wrs, eps)

    A6 = agg(z)
    Wd, bd, Wrd = p["dec_conv"]

    def tc6(a_ref, z_ref, Wd_ref, bd_ref, Wrd_ref, d_ref):
        d_ref[...] = (_dotT(a_ref[...], Wd_ref[...]) + bd_ref[...]
                      + _dotT(z_ref[...], Wrd_ref[...]))

    d = _tc(tc6, _shape(n, Wd.shape[0]), A6, z, Wd, r2(bd), Wrd)

    A7 = agg(d)
    h7 = conv_bn_relu(A7, d, p["dec_l1"]["conv1"], p["dec_l1"]["norm1"])
    A8 = agg(h7)
    b3n = conv_bn_add_relu(A8, h7, d, p["dec_l1"]["conv2"], p["dec_l1"]["norm2"])

    A9 = agg(b3n)
    h9 = conv_bn_relu(A9, b3n, p["dec_l2"]["conv1"], p["dec_l2"]["norm1"])
    A10 = agg(h9)

    Wf2l, bf2, Wf2r = p["dec_l2"]["conv2"]
    gf2, bef2 = p["dec_l2"]["norm2"]
    Wf3l, bf3, Wf3r = p["dec_l2"]["conv3"]
    gf3, bef3 = p["dec_l2"]["norm3"]

    def tc10(a10_ref, a9_ref, h9_ref, b3_ref, x_ref, kl_ref,
             W2l_ref, b2_ref, W2r_ref, g2_ref, be2_ref,
             W3l_ref, b3w_ref, W3r_ref, g3_ref, be3_ref,
             loss_ref, rl_ref):
        h2 = (_dotT(a10_ref[...], W2l_ref[...]) + b2_ref[...]
              + _dotT(h9_ref[...], W2r_ref[...]))
        h2 = _bn(h2, g2_ref[...], be2_ref[...])
        x3 = (_dotT(a9_ref[...], W3l_ref[...]) + b3w_ref[...]
              + _dotT(b3_ref[...], W3r_ref[...]))
        x3 = _bn(x3, g3_ref[...], be3_ref[...])
        dout = jax.nn.relu(h2 + x3)
        recon = jax.nn.sigmoid(dout)
        rc = jnp.clip(recon, 1e-7, 1.0 - 1e-7)
        xv = x_ref[...]
        rl = -jnp.mean(xv * jnp.log(rc) + (1.0 - xv) * jnp.log(1.0 - rc))
        rl_ref[...] = rl.reshape(1, 1)
        loss_ref[...] = rl.reshape(1, 1) + 0.2 * kl_ref[...]

    loss, recon_loss = _tc(
        tc10, [_shape(1, 1), _shape(1, 1)],
        A10, A9, h9, b3n, x, kl,
        Wf2l, r2(bf2), Wf2r, r2(gf2), r2(bef2),
        Wf3l, r2(bf3), Wf3r, r2(gf3), r2(bef3))

    return (loss.reshape(()), recon_loss.reshape(()), kl.reshape(()))


# speedup vs baseline: 1.0803x; 1.0481x over previous
import functools

import jax
import jax.numpy as jnp
from jax import lax
from jax.experimental import pallas as pl
from jax.experimental.pallas import tpu as pltpu
from jax.experimental.pallas import tpu_sc as plsc

NC = 2
NS = 16
NW = NC * NS
CHUNK = 128
N_NODES = 10000
NPAD = 10016
RPS = NPAD // NS
SPMEM_WORDS = 2_000_000


@functools.cache
def _agg_kernel(C: int, nchunks: int, dtype=jnp.float32):
    elem_words = 1 if dtype == jnp.float32 else 0.5
    per_tile = (SPMEM_WORDS - int(NPAD * C * elem_words)) // NS \
        - 2 * nchunks * CHUNK
    NBUF = max(2, min(6, int(per_tile // (CHUNK * C * elem_words))))
    mesh = plsc.VectorSubcoreMesh(
        core_axis_name="c", subcore_axis_name="s", num_cores=NC, num_subcores=NS
    )

    @functools.partial(
        pl.kernel,
        out_type=jax.ShapeDtypeStruct((NC, NPAD, C), dtype),
        mesh=mesh,
        scratch_types=[
            pltpu.VMEM((nchunks, CHUNK), jnp.int32),
            pltpu.VMEM((nchunks, CHUNK), jnp.int32),
            pltpu.VMEM((NBUF, CHUNK, C), dtype),
            pltpu.VMEM_SHARED((NPAD, C), dtype),
        ] + [pltpu.SemaphoreType.DMA] * (2 * NBUF),
        compiler_params=pltpu.CompilerParams(use_tc_tiling_on_sc=False),
    )
    def agg(table, srci, dsti, zrows, out, idx_s, idx_d, rows_v, acc, *sems):
        gsems = sems[:NBUF]
        ssems = sems[NBUF:]
        c = lax.axis_index("c")
        s = lax.axis_index("s")
        w = c * NS + s

        pltpu.sync_copy(srci.at[w], idx_s)
        gd = {}
        sd = {}
        for k in range(min(NBUF, nchunks)):
            gd[k] = pltpu.async_copy(
                table.at[idx_s.at[k]], rows_v.at[k % NBUF], gsems[k % NBUF])
        pltpu.sync_copy(dsti.at[w], idx_d)
        pltpu.sync_copy(zrows, acc.at[pl.ds(s * RPS, RPS)])
        plsc.subcore_barrier()

        for k in range(nchunks):
            b = k % NBUF
            gd[k].wait()
            sd[k] = pltpu.async_copy(
                rows_v.at[b], acc.at[idx_d.at[k]], ssems[b], add=True)
            j = k - 1 + NBUF
            if k >= 1 and j < nchunks:
                sd[k - 1].wait()
                gd[j] = pltpu.async_copy(
                    table.at[idx_s.at[j]], rows_v.at[(k - 1) % NBUF],
                    gsems[(k - 1) % NBUF])
        for k in range(max(0, nchunks - NBUF), nchunks):
            sd[k].wait()

        plsc.subcore_barrier()
        pltpu.sync_copy(
            acc.at[pl.ds(s * RPS, RPS)], out.at[c, pl.ds(s * RPS, RPS)]
        )

    return agg


def _aggregate(table, srci, dsti, nchunks):
    C = table.shape[1]
    zrows = jnp.zeros((RPS, C), table.dtype)
    return _agg_kernel(C, nchunks, table.dtype)(table, srci, dsti, zrows)


def _dotT(a, w):
    return lax.dot_general(
        a, w, (((1,), (1,)), ((), ())),
        precision=lax.Precision.HIGHEST, preferred_element_type=jnp.float32,
    )


def _bn(h, g, b):
    m = jnp.mean(h, axis=0, keepdims=True)
    v = jnp.mean((h - m) ** 2, axis=0, keepdims=True)
    return (h - m) / jnp.sqrt(v + 1e-5) * g + b


def _tc(body, out_shape, *args):
    return pl.pallas_call(
        body, out_shape=out_shape,
        compiler_params=pltpu.CompilerParams(
            vmem_limit_bytes=100 * 1024 * 1024),
    )(*args)


def _shape(*s):
    return jax.ShapeDtypeStruct(s, jnp.float32)


_FBLK = 1000


def _finalize1(p96, p64, in_c):
    def body(p96_ref, p64_ref, a_ref, rdeg_ref):
        pa = p96_ref[0].astype(jnp.float32)
        pb = p96_ref[1].astype(jnp.float32)
        deg = pa[:, 64:65] + pb[:, 64:65]
        rdeg = 1.0 / jnp.maximum(deg, 1.0)
        rdeg_ref[...] = rdeg
        a_ref[:, :64] = (pa[:, :64] + pb[:, :64]) * rdeg
        a_ref[:, 64:] = (p64_ref[0].astype(jnp.float32)
                         + p64_ref[1].astype(jnp.float32)) * rdeg

    return pl.pallas_call(
        body,
        grid=(N_NODES // _FBLK,),
        in_specs=[pl.BlockSpec((2, _FBLK, 96), lambda i: (0, i, 0)),
                  pl.BlockSpec((2, _FBLK, 64), lambda i: (0, i, 0))],
        out_specs=[pl.BlockSpec((_FBLK, in_c), lambda i: (i, 0)),
                   pl.BlockSpec((_FBLK, 1), lambda i: (i, 0))],
        out_shape=[_shape(N_NODES, in_c), _shape(N_NODES, 1)],
    )(p96, p64)


def _finalize(pAs, rdeg):
    widths = [p.shape[2] for p in pAs]
    C = sum(widths)

    def body(*refs):
        p_refs = refs[:len(pAs)]
        rdeg_ref = refs[len(pAs)]
        a_ref = refs[len(pAs) + 1]
        rd = rdeg_ref[...]
        off = 0
        for p_ref, w in zip(p_refs, widths):
            a_ref[:, off:off + w] = (p_ref[0].astype(jnp.float32)
                                     + p_ref[1].astype(jnp.float32)) * rd
            off += w

    return pl.pallas_call(
        body,
        grid=(N_NODES // _FBLK,),
        in_specs=[pl.BlockSpec((2, _FBLK, w), lambda i: (0, i, 0))
                  for w in widths]
        + [pl.BlockSpec((_FBLK, 1), lambda i: (i, 0))],
        out_specs=pl.BlockSpec((_FBLK, C), lambda i: (i, 0)),
        out_shape=_shape(N_NODES, C),
    )(*pAs, rdeg)


def _agg_from(p_refs, rd, widths=None):
    parts = []
    for i, pr in enumerate(p_refs):
        w = pr.shape[2] if widths is None else widths[i]
        parts.append(pr[0, :N_NODES, :w].astype(jnp.float32)
                     + pr[1, :N_NODES, :w].astype(jnp.float32))
    a = parts[0] if len(parts) == 1 else jnp.concatenate(parts, axis=1)
    return a * rd


def kernel(x, edge_index, params):
    n = x.shape[0]
    e = edge_index.shape[1]
    in_c = x.shape[1]
    assert n == N_NODES

    ei = edge_index.astype(jnp.int32)
    nchunks = -(-e // (NW * CHUNK))
    epad = NW * nchunks * CHUNK
    srci = jnp.concatenate(
        [ei[0], jnp.zeros((epad - e,), jnp.int32)]).reshape(NW, nchunks, CHUNK)
    dsti = jnp.concatenate(
        [ei[1], jnp.full((epad - e,), n, jnp.int32)]).reshape(NW, nchunks, CHUNK)

    def agg(table):
        C = table.shape[1]
        t16 = table.astype(jnp.bfloat16)
        return [_aggregate(t16[:, i:min(i + 64, C)], srci, dsti, nchunks)
                for i in range(0, C, 64)]

    p = params
    eps = jax.random.normal(jax.random.key(42), (n, p["gcn_mean"][0].shape[0]),
                            dtype=jnp.float32)

    def r2(v):
        return v.reshape(1, -1)

    xb = x.astype(jnp.bfloat16)
    t96 = jnp.concatenate(
        [xb[:, :64], jnp.ones((n, 32), jnp.bfloat16)], axis=1)
    p96 = _aggregate(t96, srci, dsti, nchunks)
    p64 = _aggregate(xb[:, 64:in_c], srci, dsti, nchunks)
    A1, rdeg = _finalize1(p96, p64, in_c)

    def conv_bn_relu(ps, xin, sage_p, bn_p, widths=None):
        k = len(ps)
        Wl_, bl_, Wr_ = sage_p
        g_, b_ = bn_p

        def body(*refs):
            p_refs = refs[:k]
            (rd_ref, xin_ref, Wl_ref, bl_ref, Wr_ref,
             g_ref, b_ref, out_ref) = refs[k:]
            a = _agg_from(p_refs, rd_ref[...], widths)
            h = (_dotT(a, Wl_ref[...]) + bl_ref[...]
                 + _dotT(xin_ref[...], Wr_ref[...]))
            out_ref[...] = jax.nn.relu(_bn(h, g_ref[...], b_ref[...]))

        return _tc(body, _shape(n, Wl_.shape[0]),
                   *ps, rdeg, xin, Wl_, r2(bl_), Wr_, r2(g_), r2(b_))

    def conv_bn_add_relu(ps, xin, skip, sage_p, bn_p):
        k = len(ps)
        Wl_, bl_, Wr_ = sage_p
        g_, b_ = bn_p

        def body(*refs):
            p_refs = refs[:k]
            (rd_ref, xin_ref, skip_ref, Wl_ref, bl_ref, Wr_ref,
             g_ref, b_ref, out_ref) = refs[k:]
            a = _agg_from(p_refs, rd_ref[...], None)
            h = (_dotT(a, Wl_ref[...]) + bl_ref[...]
                 + _dotT(xin_ref[...], Wr_ref[...]))
            out_ref[...] = jax.nn.relu(
                _bn(h, g_ref[...], b_ref[...]) + skip_ref[...])

        return _tc(body, _shape(n, Wl_.shape[0]),
                   *ps, rdeg, xin, skip, Wl_, r2(bl_), Wr_, r2(g_), r2(b_))

    Wl, bl, Wr = p["enc_l1"]["conv1"]
    hid = Wl.shape[0]
    def tc1(a_ref, xin_ref, Wl_ref, bl_ref, Wr_ref, g_ref, b_ref, out_ref):
        h = (_dotT(a_ref[...], Wl_ref[...]) + bl_ref[...]
             + _dotT(xin_ref[...], Wr_ref[...]))
        out_ref[...] = jax.nn.relu(_bn(h, g_ref[...], b_ref[...]))

    g1_, b1g_ = p["enc_l1"]["norm1"]
    h1 = _tc(tc1, _shape(n, hid), A1, x, Wl, r2(bl), Wr, r2(g1_), r2(b1g_))

    def tc2(a2_ref, a1_ref, rd_ref, h1_ref, x_ref,
            W2l_ref, b2_ref, W2r_ref, g2_ref, be2_ref,
            W3l_ref, b3_ref, W3r_ref, g3_ref, be3_ref, out_ref):
        a2 = _agg_from([a2_ref], rd_ref[...])
        h2 = (_dotT(a2, W2l_ref[...]) + b2_ref[...]
              + _dotT(h1_ref[...], W2r_ref[...]))
        h2 = _bn(h2, g2_ref[...], be2_ref[...])
        x3 = (_dotT(a1_ref[...], W3l_ref[...]) + b3_ref[...]
              + _dotT(x_ref[...], W3r_ref[...]))
        x3 = _bn(x3, g3_ref[...], be3_ref[...])
        out_ref[...] = jax.nn.relu(h2 + x3)

    pA2 = agg(h1)
    W2l, b2, W2r = p["enc_l1"]["conv2"]
    g2, be2 = p["enc_l1"]["norm2"]
    W3l, b3, W3r = p["enc_l1"]["conv3"]
    g3, be3 = p["enc_l1"]["norm3"]
    b1 = _tc(tc2, _shape(n, hid), pA2[0], A1, rdeg, h1, x,
             W2l, r2(b2), W2r, r2(g2), r2(be2),
             W3l, r2(b3), W3r, r2(g3), r2(be3))

    pA3 = agg(b1)
    h3 = conv_bn_relu(pA3, b1, p["enc_l2"]["conv1"], p["enc_l2"]["norm1"])
    pA4 = agg(h3)
    b2n = conv_bn_add_relu(pA4, h3, b1, p["enc_l2"]["conv2"],
                           p["enc_l2"]["norm2"])

    pA5 = agg(b2n)
    Wm, bm, Wrm = p["gcn_mean"]
    Ws, bs, Wrs = p["gcn_logstd"]

    def tc5(a_ref, rd_ref, b2_ref, Wm_ref, bm_ref, Wrm_ref,
            Ws_ref, bs_ref, Wrs_ref, eps_ref, z_ref, kl_ref):
        a = _agg_from([a_ref], rd_ref[...])
        b2v = b2_ref[...]
        mean = _dotT(a, Wm_ref[...]) + bm_ref[...] + _dotT(b2v, Wrm_ref[...])
        ls_raw = _dotT(a, Ws_ref[...]) + bs_ref[...] + _dotT(b2v, Wrs_ref[...])
        ls = jnp.clip(ls_raw, -10.0, 10.0)
        z_ref[...] = mean + eps_ref[...] * jnp.exp(ls)
        kl = -0.5 * jnp.mean(1.0 + ls_raw - mean ** 2 - jnp.exp(ls_raw))
        kl_ref[...] = kl.reshape(1, 1)

    out_c = Wm.shape[0]
    z, kl = _tc(tc5, [_shape(n, out_c), _shape(1, 1)],
                pA5[0], rdeg, b2n, Wm, r2(bm), Wrm, Ws, r2(bs), Wrs, eps)

    pA6 = agg(z)
    Wd, bd, Wrd = p["dec_conv"]

    def tc6(a_ref, rd_ref, z_ref, Wd_ref, bd_ref, Wrd_ref, d_ref):
        a = _agg_from([a_ref], rd_ref[...])
        d_ref[...] = (_dotT(a, Wd_ref[...]) + bd_ref[...]
                      + _dotT(z_ref[...], Wrd_ref[...]))

    d = _tc(tc6, _shape(n, Wd.shape[0]), pA6[0], rdeg, z, Wd, r2(bd), Wrd)

    pA7 = agg(d)
    h7 = conv_bn_relu(pA7, d, p["dec_l1"]["conv1"], p["dec_l1"]["norm1"])
    pA8 = agg(h7)
    b3n = conv_bn_add_relu(pA8, h7, d, p["dec_l1"]["conv2"],
                           p["dec_l1"]["norm2"])

    pA9 = agg(b3n)
    h9 = conv_bn_relu(pA9, b3n, p["dec_l2"]["conv1"], p["dec_l2"]["norm1"])
    A9 = _finalize(pA9, rdeg)
    A10 = _finalize(agg(h9), rdeg)

    Wf2l, bf2, Wf2r = p["dec_l2"]["conv2"]
    gf2, bef2 = p["dec_l2"]["norm2"]
    Wf3l, bf3, Wf3r = p["dec_l2"]["conv3"]
    gf3, bef3 = p["dec_l2"]["norm3"]

    def tc10(a10_ref, a9_ref, h9_ref, b3_ref, x_ref, kl_ref,
             W2l_ref, b2_ref, W2r_ref, g2_ref, be2_ref,
             W3l_ref, b3w_ref, W3r_ref, g3_ref, be3_ref,
             loss_ref, rl_ref):
        h2 = (_dotT(a10_ref[...], W2l_ref[...]) + b2_ref[...]
              + _dotT(h9_ref[...], W2r_ref[...]))
        h2 = _bn(h2, g2_ref[...], be2_ref[...])
        x3 = (_dotT(a9_ref[...], W3l_ref[...]) + b3w_ref[...]
              + _dotT(b3_ref[...], W3r_ref[...]))
        x3 = _bn(x3, g3_ref[...], be3_ref[...])
        dout = jax.nn.relu(h2 + x3)
        recon = jax.nn.sigmoid(dout)
        rc = jnp.clip(recon, 1e-7, 1.0 - 1e-7)
        xv = x_ref[...]
        rl = -jnp.mean(xv * jnp.log(rc) + (1.0 - xv) * jnp.log(1.0 - rc))
        rl_ref[...] = rl.reshape(1, 1)
        loss_ref[...] = rl.reshape(1, 1) + 0.2 * kl_ref[...]

    loss, recon_loss = _tc(
        tc10, [_shape(1, 1), _shape(1, 1)],
        A10, A9, h9, b3n, x, kl,
        Wf2l, r2(bf2), Wf2r, r2(gf2), r2(bef2),
        Wf3l, r2(bf3), Wf3r, r2(gf3), r2(bef3))

    return (loss.reshape(()), recon_loss.reshape(()), kl.reshape(()))
